# Initial kernel scaffold; baseline (speedup 1.0000x reference)
import jax
import jax.numpy as jnp
from jax.experimental import pallas as pl


def kernel(xyz, x):
    raise NotImplementedError("write your pallas kernel here")


# speedup vs baseline: 14.2274x; 14.2274x over previous
import functools

import jax
import jax.numpy as jnp
from jax import lax
from jax.experimental import pallas as pl
from jax.experimental.pallas import tpu as pltpu

_GROUP = 2048
_N = 16384
_K = 24
_BIG = 2**30
_QT = 64


def _fps_kernel(xs_ref, ys_ref, zs_ref, idx_ref, lx_ref, ly_ref, lz_ref):
    xs = xs_ref[...]
    ys = ys_ref[...]
    zs = zs_ref[...]
    lin = (lax.broadcasted_iota(jnp.int32, (16, 8, 128), 0) * 1024
           + lax.broadcasted_iota(jnp.int32, (16, 8, 128), 1) * 128
           + lax.broadcasted_iota(jnp.int32, (16, 8, 128), 2))
    tlin = (lax.broadcasted_iota(jnp.int32, (1, 8, 128), 1) * 128
            + lax.broadcasted_iota(jnp.int32, (1, 8, 128), 2))
    slot = (lax.broadcasted_iota(jnp.int32, (16, 128), 0) * 128
            + lax.broadcasted_iota(jnp.int32, (16, 128), 1))

    def read_pt(li):
        g = li // 1024
        msk = tlin == (li - g * 1024)
        px = jnp.sum(jnp.where(msk, xs_ref[pl.ds(g, 1), :, :], 0.0))
        py = jnp.sum(jnp.where(msk, ys_ref[pl.ds(g, 1), :, :], 0.0))
        pz = jnp.sum(jnp.where(msk, zs_ref[pl.ds(g, 1), :, :], 0.0))
        return px, py, pz

    def body(i, carry):
        dist, li, iacc, xacc, yacc, zacc = carry
        px, py, pz = read_pt(li)
        sel = slot == (i - 1)
        iacc = jnp.where(sel, li, iacc)
        xacc = jnp.where(sel, px, xacc)
        yacc = jnp.where(sel, py, yacc)
        zacc = jnp.where(sel, pz, zacc)
        dx = xs - px
        dy = ys - py
        dz = zs - pz
        d = dx * dx + dy * dy + dz * dz
        dist = jnp.minimum(dist, d)
        m = jnp.max(dist)
        cand = jnp.where(dist == m, lin, _BIG)
        li = jnp.min(cand)
        return dist, li, iacc, xacc, yacc, zacc

    dist0 = jnp.full((16, 8, 128), jnp.inf, dtype=jnp.float32)
    iacc0 = jnp.zeros((16, 128), jnp.int32)
    facc0 = jnp.zeros((16, 128), jnp.float32)
    dist, li, iacc, xacc, yacc, zacc = lax.fori_loop(
        1, _GROUP, body, (dist0, jnp.int32(0), iacc0, facc0, facc0, facc0))
    px, py, pz = read_pt(li)
    sel = slot == (_GROUP - 1)
    idx_ref[...] = jnp.where(sel, li, iacc)
    lx_ref[...] = jnp.where(sel, px, xacc)
    ly_ref[...] = jnp.where(sel, py, yacc)
    lz_ref[...] = jnp.where(sel, pz, zacc)


def _run_fps(pts):
    xs = pts[:, 0].reshape(16, 8, 128)
    ys = pts[:, 1].reshape(16, 8, 128)
    zs = pts[:, 2].reshape(16, 8, 128)
    out_shape = (
        jax.ShapeDtypeStruct((16, 128), jnp.int32),
        jax.ShapeDtypeStruct((16, 128), jnp.float32),
        jax.ShapeDtypeStruct((16, 128), jnp.float32),
        jax.ShapeDtypeStruct((16, 128), jnp.float32),
    )
    idx, lx, ly, lz = pl.pallas_call(_fps_kernel, out_shape=out_shape)(xs, ys, zs)
    fps_idx = idx.reshape(_GROUP)
    lc = jnp.stack([lx.reshape(_GROUP), ly.reshape(_GROUP), lz.reshape(_GROUP)],
                   axis=-1)
    return fps_idx, lc


def _knn_kernel(q_ref, pt_ref, out_ref):
    q = q_ref[...]
    pt = pt_ref[...]
    m = jnp.dot(q, pt, preferred_element_type=jnp.float32)
    pn = (pt[0:1, :] * pt[0:1, :] + pt[1:2, :] * pt[1:2, :]
          + pt[2:3, :] * pt[2:3, :])
    qn = (q[:, 0:1] * q[:, 0:1] + q[:, 1:2] * q[:, 1:2]
          + q[:, 2:3] * q[:, 2:3])
    dist = -2.0 * m
    dist = dist + qn
    dist = dist + pn
    cols = lax.broadcasted_iota(jnp.int32, (_QT, _N), 1)
    for j in range(_K):
        mn = jnp.min(dist, axis=1, keepdims=True)
        cand = jnp.where(dist == mn, cols, _BIG)
        idx = jnp.min(cand, axis=1, keepdims=True)
        out_ref[:, j:j + 1] = idx
        dist = jnp.where(cols == idx, jnp.inf, dist)


def _run_knn(lc, pts):
    q = jnp.concatenate([lc, jnp.zeros((_GROUP, 5), jnp.float32)], axis=1)
    pt = jnp.concatenate([pts.T, jnp.zeros((5, _N), jnp.float32)], axis=0)
    grid = (_GROUP // _QT,)
    knn_idx = pl.pallas_call(
        _knn_kernel,
        grid=grid,
        in_specs=[
            pl.BlockSpec((_QT, 8), lambda i: (i, 0)),
            pl.BlockSpec((8, _N), lambda i: (0, 0)),
        ],
        out_specs=pl.BlockSpec((_QT, _K), lambda i: (i, 0)),
        out_shape=jax.ShapeDtypeStruct((_GROUP, _K), jnp.int32),
    )(q, pt)
    return knn_idx


def kernel(xyz, x):
    B, N, C = x.shape
    pts = xyz[0]
    fps_idx, lc = _run_fps(pts)
    lc_xyz = lc.reshape(1, _GROUP, 3)
    knn_idx = _run_knn(lc, pts)
    lc_x = x[0][fps_idx].reshape(1, _GROUP, C)
    knn_xyz = pts[knn_idx].reshape(1, _GROUP, _K, 3)
    knn_x = x[0][knn_idx].reshape(1, _GROUP, _K, C)
    return (lc_xyz, lc_x, knn_xyz, knn_x)


# speedup vs baseline: 15.5968x; 1.0963x over previous
import functools

import jax
import jax.numpy as jnp
from jax import lax
from jax.experimental import pallas as pl
from jax.experimental.pallas import tpu as pltpu
from jax.experimental.pallas import tpu_sc as plsc

_GROUP = 2048
_N = 16384
_K = 24
_BIG = 2**30
_QT = 64


def _fps_kernel(xs_ref, ys_ref, zs_ref, idx_ref, lx_ref, ly_ref, lz_ref):
    xs = xs_ref[...]
    ys = ys_ref[...]
    zs = zs_ref[...]
    lin = (lax.broadcasted_iota(jnp.int32, (16, 8, 128), 0) * 1024
           + lax.broadcasted_iota(jnp.int32, (16, 8, 128), 1) * 128
           + lax.broadcasted_iota(jnp.int32, (16, 8, 128), 2))
    tlin = (lax.broadcasted_iota(jnp.int32, (1, 8, 128), 1) * 128
            + lax.broadcasted_iota(jnp.int32, (1, 8, 128), 2))
    slot = (lax.broadcasted_iota(jnp.int32, (16, 128), 0) * 128
            + lax.broadcasted_iota(jnp.int32, (16, 128), 1))

    def read_pt(li):
        g = li // 1024
        msk = tlin == (li - g * 1024)
        px = jnp.sum(jnp.where(msk, xs_ref[pl.ds(g, 1), :, :], 0.0))
        py = jnp.sum(jnp.where(msk, ys_ref[pl.ds(g, 1), :, :], 0.0))
        pz = jnp.sum(jnp.where(msk, zs_ref[pl.ds(g, 1), :, :], 0.0))
        return px, py, pz

    def body(i, carry):
        dist, li, iacc, xacc, yacc, zacc = carry
        px, py, pz = read_pt(li)
        sel = slot == (i - 1)
        iacc = jnp.where(sel, li, iacc)
        xacc = jnp.where(sel, px, xacc)
        yacc = jnp.where(sel, py, yacc)
        zacc = jnp.where(sel, pz, zacc)
        dx = xs - px
        dy = ys - py
        dz = zs - pz
        d = dx * dx + dy * dy + dz * dz
        dist = jnp.minimum(dist, d)
        m = jnp.max(dist)
        cand = jnp.where(dist == m, lin, _BIG)
        li = jnp.min(cand)
        return dist, li, iacc, xacc, yacc, zacc

    dist0 = jnp.full((16, 8, 128), jnp.inf, dtype=jnp.float32)
    iacc0 = jnp.zeros((16, 128), jnp.int32)
    facc0 = jnp.zeros((16, 128), jnp.float32)
    dist, li, iacc, xacc, yacc, zacc = lax.fori_loop(
        1, _GROUP, body, (dist0, jnp.int32(0), iacc0, facc0, facc0, facc0))
    px, py, pz = read_pt(li)
    sel = slot == (_GROUP - 1)
    idx_ref[...] = jnp.where(sel, li, iacc)
    lx_ref[...] = jnp.where(sel, px, xacc)
    ly_ref[...] = jnp.where(sel, py, yacc)
    lz_ref[...] = jnp.where(sel, pz, zacc)


def _run_fps(pts):
    xs = pts[:, 0].reshape(16, 8, 128)
    ys = pts[:, 1].reshape(16, 8, 128)
    zs = pts[:, 2].reshape(16, 8, 128)
    out_shape = (
        jax.ShapeDtypeStruct((16, 128), jnp.int32),
        jax.ShapeDtypeStruct((16, 128), jnp.float32),
        jax.ShapeDtypeStruct((16, 128), jnp.float32),
        jax.ShapeDtypeStruct((16, 128), jnp.float32),
    )
    idx, lx, ly, lz = pl.pallas_call(_fps_kernel, out_shape=out_shape)(xs, ys, zs)
    fps_idx = idx.reshape(_GROUP)
    lc = jnp.stack([lx.reshape(_GROUP), ly.reshape(_GROUP), lz.reshape(_GROUP)],
                   axis=-1)
    return fps_idx, lc


def _knn_kernel(q_ref, pt_ref, out_ref):
    q = q_ref[...]
    pt = pt_ref[...]
    m = jnp.dot(q, pt, preferred_element_type=jnp.float32)
    pn = (pt[0:1, :] * pt[0:1, :] + pt[1:2, :] * pt[1:2, :]
          + pt[2:3, :] * pt[2:3, :])
    qn = (q[:, 0:1] * q[:, 0:1] + q[:, 1:2] * q[:, 1:2]
          + q[:, 2:3] * q[:, 2:3])
    dist = -2.0 * m
    dist = dist + qn
    dist = dist + pn
    cols = lax.broadcasted_iota(jnp.int32, (_QT, _N), 1)
    for j in range(_K):
        mn = jnp.min(dist, axis=1, keepdims=True)
        cand = jnp.where(dist == mn, cols, _BIG)
        idx = jnp.min(cand, axis=1, keepdims=True)
        out_ref[:, j:j + 1] = idx
        dist = jnp.where(cols == idx, jnp.inf, dist)


def _run_knn(lc, pts):
    q = jnp.concatenate([lc, jnp.zeros((_GROUP, 5), jnp.float32)], axis=1)
    pt = jnp.concatenate([pts.T, jnp.zeros((5, _N), jnp.float32)], axis=0)
    grid = (_GROUP // _QT,)
    knn_idx = pl.pallas_call(
        _knn_kernel,
        grid=grid,
        in_specs=[
            pl.BlockSpec((_QT, 8), lambda i: (i, 0)),
            pl.BlockSpec((8, _N), lambda i: (0, 0)),
        ],
        out_specs=pl.BlockSpec((_QT, _K), lambda i: (i, 0)),
        out_shape=jax.ShapeDtypeStruct((_GROUP, _K), jnp.int32),
    )(q, pt)
    return knn_idx


_NW = 32
_RX = (_GROUP + _GROUP * _K) // _NW
_RK = (_GROUP * _K) // _NW
_CH = 128


def _sc_gather_kernel(x_hbm, xyzp_hbm, idx_hbm, out_x, out_k,
                      idxv, idxk, buf, bufk, sem):
    wid = lax.axis_index("s") * 2 + lax.axis_index("c")
    bx = wid * _RX
    bk = wid * _RK
    pltpu.sync_copy(idx_hbm.at[pl.ds(bx, _RX)], idxv)
    pltpu.sync_copy(idx_hbm.at[pl.ds(_GROUP + bk, _RK)], idxk)
    nfull = _RX // _CH
    tail = _RX - nfull * _CH
    for c in range(nfull):
        pltpu.async_copy(x_hbm.at[idxv.at[pl.ds(c * _CH, _CH)]], buf, sem).wait()
        pltpu.sync_copy(buf, out_x.at[pl.ds(bx + c * _CH, _CH)])
    pltpu.async_copy(x_hbm.at[idxv.at[pl.ds(nfull * _CH, tail)]],
                     buf.at[pl.ds(0, tail)], sem).wait()
    pltpu.sync_copy(buf.at[pl.ds(0, tail)],
                    out_x.at[pl.ds(bx + nfull * _CH, tail)])
    for c in range(_RK // _CH):
        pltpu.async_copy(xyzp_hbm.at[idxk.at[pl.ds(c * _CH, _CH)]], bufk,
                         sem).wait()
        pltpu.sync_copy(bufk, out_k.at[pl.ds(bk + c * _CH, _CH)])


def _run_gathers(x0, xyzp, idx_all, C):
    mesh = plsc.VectorSubcoreMesh(core_axis_name="c", subcore_axis_name="s")
    gk = functools.partial(
        pl.kernel,
        mesh=mesh,
        out_type=[
            jax.ShapeDtypeStruct((_GROUP + _GROUP * _K, C), jnp.float32),
            jax.ShapeDtypeStruct((_GROUP * _K, 128), jnp.float32),
        ],
        scratch_types=[
            pltpu.VMEM((_RX,), jnp.int32),
            pltpu.VMEM((_RK,), jnp.int32),
            pltpu.VMEM((_CH, C), jnp.float32),
            pltpu.VMEM((_CH, 128), jnp.float32),
            pltpu.SemaphoreType.DMA,
        ],
    )(_sc_gather_kernel)
    return gk(x0, xyzp, idx_all)


def kernel(xyz, x):
    B, N, C = x.shape
    pts = xyz[0]
    fps_idx, lc = _run_fps(pts)
    lc_xyz = lc.reshape(1, _GROUP, 3)
    knn_idx = _run_knn(lc, pts)
    idx_all = jnp.concatenate([fps_idx, knn_idx.reshape(-1)])
    xyzp = jnp.pad(pts, ((0, 0), (0, 125)))
    out_x, out_k = _run_gathers(x[0], xyzp, idx_all, C)
    lc_x = out_x[:_GROUP].reshape(1, _GROUP, C)
    knn_x = out_x[_GROUP:].reshape(1, _GROUP, _K, C)
    knn_xyz = out_k[:, :3].reshape(1, _GROUP, _K, 3)
    return (lc_xyz, lc_x, knn_xyz, knn_x)
